# baseline (speedup 1.0000x reference)
import functools

import jax
import jax.numpy as jnp
from jax import lax
from jax.experimental import pallas as pl
from jax.experimental.pallas import tpu as pltpu
from jax.experimental.pallas import tpu_sc as plsc

_NC = 2
_NS = 16
_NW = _NC * _NS
_CH = 40
_CT = 250
_N = 10000
_D = 128
_RPT = _N // _NS

_C1 = 0.3932355018112294
_C3 = -0.05760769359106874
_C5 = 0.005889678243760147
_C7 = -0.00031323817551585746
_C9 = 6.549354500097471e-06


def _xl_body(x_ref, w_ref, b_ref, o_ref):
    o_ref[...] = lax.dot_general(
        x_ref[...], w_ref[...], (((1,), (1,)), ((), ())),
        preferred_element_type=jnp.float32) + b_ref[...]


def _xl_call(x, w, b):
    n, d = x.shape
    blk = 2000
    return pl.pallas_call(
        _xl_body,
        grid=(n // blk,),
        in_specs=[
            pl.BlockSpec((blk, d), lambda i: (i, 0)),
            pl.BlockSpec((d, d), lambda i: (0, 0)),
            pl.BlockSpec((1, d), lambda i: (0, 0)),
        ],
        out_specs=pl.BlockSpec((blk, d), lambda i: (i, 0)),
        out_shape=jax.ShapeDtypeStruct((n, d), jnp.float32),
    )(x, w, b)


_WS = 1.0 / 16777216.0


def _sc_body(xl, estk, attr, outp, cntp,
             est0, est1, didx0, didx1, dstr, wspl, g0, g1, a0, a1,
             cbuf, out_sh, cnt_sh, sem_i, sem_g, sem_a, sem_s):
    cid = lax.axis_index("c")
    sid = lax.axis_index("s")
    wid = sid * _NC + cid
    cbase = wid * _CT

    zero16 = jnp.zeros((16,), jnp.float32)
    ones16 = jnp.ones((16,), jnp.float32)

    def _zg(i, c):
        for k in range(8):
            g0[i, pl.ds(k * 16, 16)] = zero16
            cbuf[i, pl.ds(k * 16, 16)] = zero16
        return c
    lax.fori_loop(0, _CH, _zg, 0)
    for t in range(_RPT // _CH):
        pltpu.sync_copy(g0, out_sh.at[pl.ds(sid * _RPT + t * _CH, _CH)])
    _rem = _RPT % _CH
    if _rem:
        pltpu.sync_copy(
            g0.at[pl.ds(0, _rem)],
            out_sh.at[pl.ds(sid * _RPT + (_RPT // _CH) * _CH, _rem)])
    @pl.when(sid < 10)
    def _zc():
        for t in range(125 // _CH):
            pltpu.sync_copy(cbuf, cnt_sh.at[pl.ds(sid * 125 + t * _CH, _CH)])
        _crem = 125 % _CH
        if _crem:
            pltpu.sync_copy(
                cbuf.at[pl.ds(0, _crem)],
                cnt_sh.at[pl.ds(sid * 125 + (125 // _CH) * _CH, _crem)])

    pltpu.async_copy(estk.at[cbase], est0, sem_i)
    pltpu.make_async_copy(estk.at[cbase], est0, sem_i).wait()
    pltpu.async_copy(xl.at[est0.at[0]], g0, sem_g)
    pltpu.async_copy(attr.at[pl.ds(cbase * _CH, _CH)], a0, sem_a)
    pltpu.async_copy(estk.at[cbase + 1], est1, sem_i)

    plsc.subcore_barrier()

    def _half(s, est, est_n, g, g_n, a, a_n, didx, didx_p):
        last = _CT - 1

        for q in range((_CH + 15) // 16):
            e0 = min(q * 16, _CH - 16)
            dv16 = est[1, pl.ds(e0, 16)]
            didx[pl.ds(e0, 16)] = dv16
            dstr[pl.ds(e0, 16)] = dv16 >> 3
            wvf = est[2, pl.ds(e0, 16)].astype(jnp.float32) * _WS
            for i in range(16):
                wspl[e0 + i, :] = jnp.full((16,), wvf[i], jnp.float32)
                off = (dv16[i] & 7) * 16
                cbuf[e0 + i, pl.ds(off, 16)] = ones16

        pltpu.make_async_copy(xl.at[est.at[0]], g, sem_g).wait()

        @pl.when(s + 2 <= last)
        def _pf2():
            pltpu.async_copy(estk.at[cbase + s + 2], est, sem_i)

        @pl.when(s > 0)
        def _ws():
            pltpu.make_async_copy(a_n, out_sh.at[didx_p], sem_s).wait()

        @pl.when(s < last)
        def _pf():
            pltpu.make_async_copy(estk.at[cbase], est_n, sem_i).wait()
            pltpu.async_copy(xl.at[est_n.at[0]], g_n, sem_g)
            pltpu.async_copy(attr.at[pl.ds((cbase + s + 1) * _CH, _CH)],
                             a_n, sem_a)

        pltpu.make_async_copy(attr.at[pl.ds(0, _CH)], a, sem_a).wait()

        @plsc.parallel_loop(0, _CH, 1, unroll=16)
        def _edge(e):
            wrow = wspl[e, :]
            for k in range(8):
                sl = pl.ds(k * 16, 16)
                xv = g[e, sl] + a[e, sl]
                cv = jnp.minimum(jnp.maximum(xv, -4.0), 4.0)
                z = cv * cv
                p5 = (((_C9 * z + _C7) * z + _C5) * z + _C3) * z + _C1
                a[e, sl] = (xv * wrow) * (0.5 + cv * p5)

        pltpu.async_copy(a, out_sh.at[didx], sem_s, add=True)
        pltpu.sync_copy(cbuf, cnt_sh.at[dstr], add=True)

        def _clr(q, c2):
            e0 = jnp.minimum(q * 16, _CH - 16)
            dvec = didx[pl.ds(e0, 16)]
            for i in range(16):
                off = (dvec[i] & 7) * 16
                cbuf[e0 + i, pl.ds(off, 16)] = zero16
            return c2
        lax.fori_loop(0, (_CH + 15) // 16, _clr, 0)

    def _pair(p, c):
        s = p * 2
        _half(s, est0, est1, g0, g1, a0, a1, didx0, didx1)
        _half(s + 1, est1, est0, g1, g0, a1, a0, didx1, didx0)
        return c
    lax.fori_loop(0, _CT // 2, _pair, 0)

    pltpu.make_async_copy(a1, out_sh.at[didx1], sem_s).wait()

    plsc.subcore_barrier()

    pltpu.sync_copy(out_sh.at[pl.ds(sid * _RPT, _RPT)], outp.at[cid, sid])

    @pl.when(sid == 0)
    def _dc():
        pltpu.sync_copy(cnt_sh, cntp.at[cid])


def _sc_call(xl, estk, attr):
    mesh = plsc.VectorSubcoreMesh(core_axis_name="c", subcore_axis_name="s")
    f = pl.kernel(
        _sc_body,
        out_type=[
            jax.ShapeDtypeStruct((_NC, _NS, _RPT, _D), jnp.float32),
            jax.ShapeDtypeStruct((_NC, _N // 8, _D), jnp.float32),
        ],
        mesh=mesh,
        scratch_types=[
            pltpu.VMEM((3, _CH), jnp.int32),
            pltpu.VMEM((3, _CH), jnp.int32),
            pltpu.VMEM((_CH,), jnp.int32),
            pltpu.VMEM((_CH,), jnp.int32),
            pltpu.VMEM((_CH,), jnp.int32),
            pltpu.VMEM((_CH, 16), jnp.float32),
            pltpu.VMEM((_CH, _D), jnp.float32),
            pltpu.VMEM((_CH, _D), jnp.float32),
            pltpu.VMEM((_CH, _D), jnp.float32),
            pltpu.VMEM((_CH, _D), jnp.float32),
            pltpu.VMEM((_CH, _D), jnp.float32),
            pltpu.VMEM_SHARED((_N, _D), jnp.float32),
            pltpu.VMEM_SHARED((_N // 8, _D), jnp.float32),
            pltpu.SemaphoreType.DMA,
            pltpu.SemaphoreType.DMA,
            pltpu.SemaphoreType.DMA,
            pltpu.SemaphoreType.DMA,
        ],
    )
    return f(xl, estk, attr)


def _fin_body(op_ref, cnt_ref, x_ref, wl_ref, bl_ref, wr_ref, o_ref):
    s = op_ref[0] + op_ref[1]
    c = cnt_ref[0, 0] + cnt_ref[0, 1]
    r = 1.0 / jnp.maximum(c, 1.0)
    t = lax.dot_general(s, wl_ref[...], (((1,), (1,)), ((), ())),
                        preferred_element_type=jnp.float32)
    u = lax.dot_general(x_ref[...], wr_ref[...], (((1,), (1,)), ((), ())),
                        preferred_element_type=jnp.float32)
    o_ref[...] = t * r[:, None] + bl_ref[...] + u


def _fin_call(outp, cnt, x, wl, bl, wr):
    n, d = x.shape
    blk = 2000
    return pl.pallas_call(
        _fin_body,
        grid=(n // blk,),
        in_specs=[
            pl.BlockSpec((_NC, blk, d), lambda i: (0, i, 0)),
            pl.BlockSpec((1, _NC, blk), lambda i: (i, 0, 0)),
            pl.BlockSpec((blk, d), lambda i: (i, 0)),
            pl.BlockSpec((d, d), lambda i: (0, 0)),
            pl.BlockSpec((1, d), lambda i: (0, 0)),
            pl.BlockSpec((d, d), lambda i: (0, 0)),
        ],
        out_specs=pl.BlockSpec((blk, d), lambda i: (i, 0)),
        out_shape=jax.ShapeDtypeStruct((n, d), jnp.float32),
    )(outp, cnt, x, wl, bl, wr)


def kernel(x, edge_index, edge_attr, edge_weight, W_lin, b_lin, W_l, b_l, W_r):
    n, d = x.shape
    src1 = edge_index[0].astype(jnp.int32).reshape(_NW * _CT, _CH)
    dst1 = edge_index[1].astype(jnp.int32).reshape(_NW * _CT, _CH)
    wq = (edge_weight.reshape(_NW * _CT, _CH) * 16777216.0).astype(jnp.int32)
    estk = jnp.stack([src1, dst1, wq], axis=1)
    xl = _xl_call(x, W_lin, b_lin.reshape(1, d))
    outp, cntp = _sc_call(xl, estk, edge_attr)
    cnt = cntp.reshape(_NC, n // 8, 8, 16)[:, :, :, 0].reshape(_NC, 5, n // 5)
    cnt = cnt.transpose(1, 0, 2)
    return _fin_call(outp.reshape(_NC, n, d), cnt, x, W_l, b_l.reshape(1, d),
                     W_r)


# speedup vs baseline: 1.5321x; 1.5321x over previous
import functools

import jax
import jax.numpy as jnp
from jax import lax
from jax.experimental import pallas as pl
from jax.experimental.pallas import tpu as pltpu
from jax.experimental.pallas import tpu_sc as plsc

_NC = 2
_NS = 16
_NW = _NC * _NS
_CH = 40
_CT = 250
_N = 10000
_D = 128
_RPT = _N // _NS

_C1 = 0.3813765833554562
_C3 = -0.04667325110762056
_C5 = 0.003267293765639234
_C7 = -8.490068491382591e-05


def _xl_body(x_ref, w_ref, b_ref, o_ref):
    o_ref[...] = lax.dot_general(
        x_ref[...], w_ref[...], (((1,), (1,)), ((), ())),
        preferred_element_type=jnp.float32) + b_ref[...]


def _xl_call(x, w, b):
    n, d = x.shape
    blk = 2000
    return pl.pallas_call(
        _xl_body,
        grid=(n // blk,),
        in_specs=[
            pl.BlockSpec((blk, d), lambda i: (i, 0)),
            pl.BlockSpec((d, d), lambda i: (0, 0)),
            pl.BlockSpec((1, d), lambda i: (0, 0)),
        ],
        out_specs=pl.BlockSpec((blk, d), lambda i: (i, 0)),
        out_shape=jax.ShapeDtypeStruct((n, d), jnp.float32),
    )(x, w, b)


_WS = 1.0 / 16777216.0


def _sc_body(xl, estk, attr, outp, cntp,
             est0, est1, didx0, didx1, dstr, wspl, g0, g1, a0, a1,
             cbuf, out_sh, cnt_sh, sem_i, sem_g, sem_a, sem_s):
    cid = lax.axis_index("c")
    sid = lax.axis_index("s")
    wid = sid * _NC + cid
    cbase = wid * _CT

    zero16 = jnp.zeros((16,), jnp.float32)
    ones16 = jnp.ones((16,), jnp.float32)

    def _zg(i, c):
        for k in range(8):
            g0[i, pl.ds(k * 16, 16)] = zero16
            cbuf[i, pl.ds(k * 16, 16)] = zero16
        return c
    lax.fori_loop(0, _CH, _zg, 0)
    for t in range(_RPT // _CH):
        pltpu.sync_copy(g0, out_sh.at[pl.ds(sid * _RPT + t * _CH, _CH)])
    _rem = _RPT % _CH
    if _rem:
        pltpu.sync_copy(
            g0.at[pl.ds(0, _rem)],
            out_sh.at[pl.ds(sid * _RPT + (_RPT // _CH) * _CH, _rem)])
    @pl.when(sid < 10)
    def _zc():
        for t in range(125 // _CH):
            pltpu.sync_copy(cbuf, cnt_sh.at[pl.ds(sid * 125 + t * _CH, _CH)])
        _crem = 125 % _CH
        if _crem:
            pltpu.sync_copy(
                cbuf.at[pl.ds(0, _crem)],
                cnt_sh.at[pl.ds(sid * 125 + (125 // _CH) * _CH, _crem)])

    pltpu.async_copy(estk.at[cbase], est0, sem_i)
    pltpu.make_async_copy(estk.at[cbase], est0, sem_i).wait()
    pltpu.async_copy(xl.at[est0.at[0]], g0, sem_g)
    pltpu.async_copy(attr.at[pl.ds(cbase * _CH, _CH)], a0, sem_a)
    pltpu.async_copy(estk.at[cbase + 1], est1, sem_i)

    plsc.subcore_barrier()

    def _half(s, est, est_n, g, g_n, a, a_n, didx, didx_p):
        last = _CT - 1

        for q in range((_CH + 15) // 16):
            e0 = min(q * 16, _CH - 16)
            dv16 = est[1, pl.ds(e0, 16)]
            didx[pl.ds(e0, 16)] = dv16
            dstr[pl.ds(e0, 16)] = dv16 >> 3
            wvf = est[2, pl.ds(e0, 16)].astype(jnp.float32) * _WS
            for i in range(16):
                wspl[e0 + i, :] = jnp.full((16,), wvf[i], jnp.float32)
                off = (dv16[i] & 7) * 16
                cbuf[e0 + i, pl.ds(off, 16)] = ones16

        pltpu.make_async_copy(xl.at[est.at[0]], g, sem_g).wait()

        @pl.when(s + 2 <= last)
        def _pf2():
            pltpu.async_copy(estk.at[cbase + s + 2], est, sem_i)

        @pl.when(s > 0)
        def _ws():
            pltpu.make_async_copy(a_n, out_sh.at[didx_p], sem_s).wait()

        @pl.when(s < last)
        def _pf():
            pltpu.make_async_copy(estk.at[cbase], est_n, sem_i).wait()
            pltpu.async_copy(xl.at[est_n.at[0]], g_n, sem_g)
            pltpu.async_copy(attr.at[pl.ds((cbase + s + 1) * _CH, _CH)],
                             a_n, sem_a)

        pltpu.make_async_copy(attr.at[pl.ds(0, _CH)], a, sem_a).wait()

        @plsc.parallel_loop(0, _CH, 1, unroll=8)
        def _edge(e):
            wrow = wspl[e, :]
            for k in range(8):
                sl = pl.ds(k * 16, 16)
                xv = g[e, sl] + a[e, sl]
                cv = jnp.minimum(jnp.maximum(xv, -4.0), 4.0)
                z = cv * cv
                p5 = ((_C7 * z + _C5) * z + _C3) * z + _C1
                a[e, sl] = (xv * wrow) * (0.5 + cv * p5)

        pltpu.async_copy(a, out_sh.at[didx], sem_s, add=True)
        pltpu.sync_copy(cbuf, cnt_sh.at[dstr], add=True)

        def _clr(q, c2):
            e0 = jnp.minimum(q * 16, _CH - 16)
            dvec = didx[pl.ds(e0, 16)]
            for i in range(16):
                off = (dvec[i] & 7) * 16
                cbuf[e0 + i, pl.ds(off, 16)] = zero16
            return c2
        lax.fori_loop(0, (_CH + 15) // 16, _clr, 0)

    def _pair(p, c):
        s = p * 2
        _half(s, est0, est1, g0, g1, a0, a1, didx0, didx1)
        _half(s + 1, est1, est0, g1, g0, a1, a0, didx1, didx0)
        return c
    lax.fori_loop(0, _CT // 2, _pair, 0)

    pltpu.make_async_copy(a1, out_sh.at[didx1], sem_s).wait()

    plsc.subcore_barrier()

    pltpu.sync_copy(out_sh.at[pl.ds(sid * _RPT, _RPT)], outp.at[cid, sid])

    @pl.when(sid == 0)
    def _dc():
        pltpu.sync_copy(cnt_sh, cntp.at[cid])


def _sc_call(xl, estk, attr):
    mesh = plsc.VectorSubcoreMesh(core_axis_name="c", subcore_axis_name="s")
    f = pl.kernel(
        _sc_body,
        out_type=[
            jax.ShapeDtypeStruct((_NC, _NS, _RPT, _D), jnp.float32),
            jax.ShapeDtypeStruct((_NC, _N // 8, _D), jnp.float32),
        ],
        mesh=mesh,
        scratch_types=[
            pltpu.VMEM((3, _CH), jnp.int32),
            pltpu.VMEM((3, _CH), jnp.int32),
            pltpu.VMEM((_CH,), jnp.int32),
            pltpu.VMEM((_CH,), jnp.int32),
            pltpu.VMEM((_CH,), jnp.int32),
            pltpu.VMEM((_CH, 16), jnp.float32),
            pltpu.VMEM((_CH, _D), jnp.float32),
            pltpu.VMEM((_CH, _D), jnp.float32),
            pltpu.VMEM((_CH, _D), jnp.float32),
            pltpu.VMEM((_CH, _D), jnp.float32),
            pltpu.VMEM((_CH, _D), jnp.float32),
            pltpu.VMEM_SHARED((_N, _D), jnp.float32),
            pltpu.VMEM_SHARED((_N // 8, _D), jnp.float32),
            pltpu.SemaphoreType.DMA,
            pltpu.SemaphoreType.DMA,
            pltpu.SemaphoreType.DMA,
            pltpu.SemaphoreType.DMA,
        ],
    )
    return f(xl, estk, attr)


def _fin_body(op_ref, cnt_ref, x_ref, wl_ref, bl_ref, wr_ref, o_ref):
    s = op_ref[0] + op_ref[1]
    c = cnt_ref[0, 0] + cnt_ref[0, 1]
    r = 1.0 / jnp.maximum(c, 1.0)
    t = lax.dot_general(s, wl_ref[...], (((1,), (1,)), ((), ())),
                        preferred_element_type=jnp.float32)
    u = lax.dot_general(x_ref[...], wr_ref[...], (((1,), (1,)), ((), ())),
                        preferred_element_type=jnp.float32)
    o_ref[...] = t * r[:, None] + bl_ref[...] + u


def _fin_call(outp, cnt, x, wl, bl, wr):
    n, d = x.shape
    blk = 2000
    return pl.pallas_call(
        _fin_body,
        grid=(n // blk,),
        in_specs=[
            pl.BlockSpec((_NC, blk, d), lambda i: (0, i, 0)),
            pl.BlockSpec((1, _NC, blk), lambda i: (i, 0, 0)),
            pl.BlockSpec((blk, d), lambda i: (i, 0)),
            pl.BlockSpec((d, d), lambda i: (0, 0)),
            pl.BlockSpec((1, d), lambda i: (0, 0)),
            pl.BlockSpec((d, d), lambda i: (0, 0)),
        ],
        out_specs=pl.BlockSpec((blk, d), lambda i: (i, 0)),
        out_shape=jax.ShapeDtypeStruct((n, d), jnp.float32),
    )(outp, cnt, x, wl, bl, wr)


def kernel(x, edge_index, edge_attr, edge_weight, W_lin, b_lin, W_l, b_l, W_r):
    n, d = x.shape
    src1 = edge_index[0].astype(jnp.int32).reshape(_NW * _CT, _CH)
    dst1 = edge_index[1].astype(jnp.int32).reshape(_NW * _CT, _CH)
    wq = (edge_weight.reshape(_NW * _CT, _CH) * 16777216.0).astype(jnp.int32)
    estk = jnp.stack([src1, dst1, wq], axis=1)
    xl = _xl_call(x, W_lin, b_lin.reshape(1, d))
    outp, cntp = _sc_call(xl, estk, edge_attr)
    cnt = cntp.reshape(_NC, n // 8, 8, 16)[:, :, :, 0].reshape(_NC, 5, n // 5)
    cnt = cnt.transpose(1, 0, 2)
    return _fin_call(outp.reshape(_NC, n, d), cnt, x, W_l, b_l.reshape(1, d),
                     W_r)


# speedup vs baseline: 1.6333x; 1.0661x over previous
import functools

import jax
import jax.numpy as jnp
from jax import lax
from jax.experimental import pallas as pl
from jax.experimental.pallas import tpu as pltpu
from jax.experimental.pallas import tpu_sc as plsc

_NC = 2
_NS = 16
_NW = _NC * _NS
_CH = 40
_CT = 250
_N = 10000
_D = 128
_RPT = _N // _NS

_C1 = 0.3517254754305821
_C3 = -0.029789938828602848
_C5 = 0.0009978543118353798


def _xl_body(x_ref, w_ref, b_ref, o_ref):
    o_ref[...] = lax.dot_general(
        x_ref[...], w_ref[...], (((1,), (1,)), ((), ())),
        preferred_element_type=jnp.float32) + b_ref[...]


def _xl_call(x, w, b):
    n, d = x.shape
    blk = 2000
    return pl.pallas_call(
        _xl_body,
        grid=(n // blk,),
        in_specs=[
            pl.BlockSpec((blk, d), lambda i: (i, 0)),
            pl.BlockSpec((d, d), lambda i: (0, 0)),
            pl.BlockSpec((1, d), lambda i: (0, 0)),
        ],
        out_specs=pl.BlockSpec((blk, d), lambda i: (i, 0)),
        out_shape=jax.ShapeDtypeStruct((n, d), jnp.float32),
    )(x, w, b)


_WS = 1.0 / 16777216.0


def _sc_body(xl, estk, attr, outp, cntp,
             est0, est1, didx0, didx1, dstr, wspl, g0, g1, a0, a1,
             cbuf, out_sh, cnt_sh, sem_i, sem_g, sem_a, sem_s):
    cid = lax.axis_index("c")
    sid = lax.axis_index("s")
    wid = sid * _NC + cid
    cbase = wid * _CT

    zero16 = jnp.zeros((16,), jnp.float32)
    ones16 = jnp.ones((16,), jnp.float32)

    def _zg(i, c):
        for k in range(8):
            g0[i, pl.ds(k * 16, 16)] = zero16
            cbuf[i, pl.ds(k * 16, 16)] = zero16
        return c
    lax.fori_loop(0, _CH, _zg, 0)
    for t in range(_RPT // _CH):
        pltpu.sync_copy(g0, out_sh.at[pl.ds(sid * _RPT + t * _CH, _CH)])
    _rem = _RPT % _CH
    if _rem:
        pltpu.sync_copy(
            g0.at[pl.ds(0, _rem)],
            out_sh.at[pl.ds(sid * _RPT + (_RPT // _CH) * _CH, _rem)])
    @pl.when(sid < 10)
    def _zc():
        for t in range(125 // _CH):
            pltpu.sync_copy(cbuf, cnt_sh.at[pl.ds(sid * 125 + t * _CH, _CH)])
        _crem = 125 % _CH
        if _crem:
            pltpu.sync_copy(
                cbuf.at[pl.ds(0, _crem)],
                cnt_sh.at[pl.ds(sid * 125 + (125 // _CH) * _CH, _crem)])

    pltpu.async_copy(estk.at[cbase], est0, sem_i)
    pltpu.make_async_copy(estk.at[cbase], est0, sem_i).wait()
    pltpu.async_copy(xl.at[est0.at[0]], g0, sem_g)
    pltpu.async_copy(attr.at[pl.ds(cbase * _CH, _CH)], a0, sem_a)
    pltpu.async_copy(estk.at[cbase + 1], est1, sem_i)

    plsc.subcore_barrier()

    def _half(s, est, est_n, g, g_n, a, a_n, didx, didx_p):
        last = _CT - 1

        for q in range((_CH + 15) // 16):
            e0 = min(q * 16, _CH - 16)
            dv16 = est[1, pl.ds(e0, 16)]
            didx[pl.ds(e0, 16)] = dv16
            dstr[pl.ds(e0, 16)] = dv16 >> 3
            wvf = est[2, pl.ds(e0, 16)].astype(jnp.float32) * _WS
            for i in range(16):
                wspl[e0 + i, :] = jnp.full((16,), wvf[i], jnp.float32)
                off = (dv16[i] & 7) * 16
                cbuf[e0 + i, pl.ds(off, 16)] = ones16

        pltpu.make_async_copy(xl.at[est.at[0]], g, sem_g).wait()

        @pl.when(s + 2 <= last)
        def _pf2():
            pltpu.async_copy(estk.at[cbase + s + 2], est, sem_i)

        @pl.when(s > 0)
        def _ws():
            pltpu.make_async_copy(a_n, out_sh.at[didx_p], sem_s).wait()

        @pl.when(s < last)
        def _pf():
            pltpu.make_async_copy(estk.at[cbase], est_n, sem_i).wait()
            pltpu.async_copy(xl.at[est_n.at[0]], g_n, sem_g)
            pltpu.async_copy(attr.at[pl.ds((cbase + s + 1) * _CH, _CH)],
                             a_n, sem_a)

        pltpu.make_async_copy(attr.at[pl.ds(0, _CH)], a, sem_a).wait()

        @plsc.parallel_loop(0, _CH, 1, unroll=8)
        def _edge(e):
            wrow = wspl[e, :]
            for k in range(8):
                sl = pl.ds(k * 16, 16)
                xv = g[e, sl] + a[e, sl]
                cv = jnp.minimum(jnp.maximum(xv, -4.0), 4.0)
                z = cv * cv
                p5 = (_C5 * z + _C3) * z + _C1
                a[e, sl] = (xv * wrow) * (0.5 + cv * p5)

        pltpu.async_copy(a, out_sh.at[didx], sem_s, add=True)
        pltpu.sync_copy(cbuf, cnt_sh.at[dstr], add=True)

        def _clr(q, c2):
            e0 = jnp.minimum(q * 16, _CH - 16)
            dvec = didx[pl.ds(e0, 16)]
            for i in range(16):
                off = (dvec[i] & 7) * 16
                cbuf[e0 + i, pl.ds(off, 16)] = zero16
            return c2
        lax.fori_loop(0, (_CH + 15) // 16, _clr, 0)

    def _pair(p, c):
        s = p * 2
        _half(s, est0, est1, g0, g1, a0, a1, didx0, didx1)
        _half(s + 1, est1, est0, g1, g0, a1, a0, didx1, didx0)
        return c
    lax.fori_loop(0, _CT // 2, _pair, 0)

    pltpu.make_async_copy(a1, out_sh.at[didx1], sem_s).wait()

    plsc.subcore_barrier()

    pltpu.sync_copy(out_sh.at[pl.ds(sid * _RPT, _RPT)], outp.at[cid, sid])

    @pl.when(sid == 0)
    def _dc():
        pltpu.sync_copy(cnt_sh, cntp.at[cid])


def _sc_call(xl, estk, attr):
    mesh = plsc.VectorSubcoreMesh(core_axis_name="c", subcore_axis_name="s")
    f = pl.kernel(
        _sc_body,
        out_type=[
            jax.ShapeDtypeStruct((_NC, _NS, _RPT, _D), jnp.float32),
            jax.ShapeDtypeStruct((_NC, _N // 8, _D), jnp.float32),
        ],
        mesh=mesh,
        scratch_types=[
            pltpu.VMEM((3, _CH), jnp.int32),
            pltpu.VMEM((3, _CH), jnp.int32),
            pltpu.VMEM((_CH,), jnp.int32),
            pltpu.VMEM((_CH,), jnp.int32),
            pltpu.VMEM((_CH,), jnp.int32),
            pltpu.VMEM((_CH, 16), jnp.float32),
            pltpu.VMEM((_CH, _D), jnp.float32),
            pltpu.VMEM((_CH, _D), jnp.float32),
            pltpu.VMEM((_CH, _D), jnp.float32),
            pltpu.VMEM((_CH, _D), jnp.float32),
            pltpu.VMEM((_CH, _D), jnp.float32),
            pltpu.VMEM_SHARED((_N, _D), jnp.float32),
            pltpu.VMEM_SHARED((_N // 8, _D), jnp.float32),
            pltpu.SemaphoreType.DMA,
            pltpu.SemaphoreType.DMA,
            pltpu.SemaphoreType.DMA,
            pltpu.SemaphoreType.DMA,
        ],
    )
    return f(xl, estk, attr)


def _fin_body(op_ref, cnt_ref, x_ref, wl_ref, bl_ref, wr_ref, o_ref):
    s = op_ref[0] + op_ref[1]
    c = cnt_ref[0, 0] + cnt_ref[0, 1]
    r = 1.0 / jnp.maximum(c, 1.0)
    t = lax.dot_general(s, wl_ref[...], (((1,), (1,)), ((), ())),
                        preferred_element_type=jnp.float32)
    u = lax.dot_general(x_ref[...], wr_ref[...], (((1,), (1,)), ((), ())),
                        preferred_element_type=jnp.float32)
    o_ref[...] = t * r[:, None] + bl_ref[...] + u


def _fin_call(outp, cnt, x, wl, bl, wr):
    n, d = x.shape
    blk = 2000
    return pl.pallas_call(
        _fin_body,
        grid=(n // blk,),
        in_specs=[
            pl.BlockSpec((_NC, blk, d), lambda i: (0, i, 0)),
            pl.BlockSpec((1, _NC, blk), lambda i: (i, 0, 0)),
            pl.BlockSpec((blk, d), lambda i: (i, 0)),
            pl.BlockSpec((d, d), lambda i: (0, 0)),
            pl.BlockSpec((1, d), lambda i: (0, 0)),
            pl.BlockSpec((d, d), lambda i: (0, 0)),
        ],
        out_specs=pl.BlockSpec((blk, d), lambda i: (i, 0)),
        out_shape=jax.ShapeDtypeStruct((n, d), jnp.float32),
    )(outp, cnt, x, wl, bl, wr)


def kernel(x, edge_index, edge_attr, edge_weight, W_lin, b_lin, W_l, b_l, W_r):
    n, d = x.shape
    src1 = edge_index[0].astype(jnp.int32).reshape(_NW * _CT, _CH)
    dst1 = edge_index[1].astype(jnp.int32).reshape(_NW * _CT, _CH)
    wq = (edge_weight.reshape(_NW * _CT, _CH) * 16777216.0).astype(jnp.int32)
    estk = jnp.stack([src1, dst1, wq], axis=1)
    xl = _xl_call(x, W_lin, b_lin.reshape(1, d))
    outp, cntp = _sc_call(xl, estk, edge_attr)
    cnt = cntp.reshape(_NC, n // 8, 8, 16)[:, :, :, 0].reshape(_NC, 5, n // 5)
    cnt = cnt.transpose(1, 0, 2)
    return _fin_call(outp.reshape(_NC, n, d), cnt, x, W_l, b_l.reshape(1, d),
                     W_r)
